# baseline (speedup 1.0000x reference)
import functools

import jax
import jax.numpy as jnp
import numpy as np
from jax import lax
from jax.experimental import pallas as pl
from jax.experimental.pallas import tpu as pltpu
from jax.experimental.pallas import tpu_sc as plsc

B = 4
P = 50
H = 200
W = 200
C = 80
R = 2048
S = 128

NC = 2
NS = 16
NW = NC * NS
RPW = R // NW
GROUPS = RPW // 16
GSZ = 16 * S * 2

HW = H * W
MAGIC = np.float32(2.0 ** 23)
INV_T = np.float32(1.0 / (S - 1))
INV_S = np.float32(1.0 / S)
RSQRT_MAGIC = np.int32(0x5F3759DF)
LOG_EPS = np.float32(np.log(np.float32(1e-12)))
NEG_INF = np.float32(-np.inf)
LOSS_W = np.float32(0.1)


def _rsqrt_f32(x):
    i = lax.bitcast_convert_type(x, jnp.int32)
    i = RSQRT_MAGIC - lax.shift_right_logical(i, 1)
    y = lax.bitcast_convert_type(i, jnp.float32)
    for _ in range(3):
        y = y * (np.float32(1.5) - np.float32(0.5) * x * y * y)
    return y


def _rint_idx(x):
    r = (x + MAGIC) - MAGIC
    r = jnp.minimum(jnp.maximum(r, np.float32(0.0)), np.float32(199.0))
    return r.astype(jnp.int32)


def _sc_body(raf_hbm, hm_hbm, bi_hbm, scl_hbm, ocl_hbm, prd_hbm,
             sx_hbm, sy_hbm, ox_hbm, oy_hbm,
             integ_hbm, so_hbm, valid_hbm,
             int_buf, idx_buf, g_buf, hidx, hval,
             uxb, uyb, integb, sob, validb, sem_in, sem_r, sem_h):
    wid = lax.axis_index("c") * NS + lax.axis_index("s")
    base = wid * RPW

    ins = (bi_hbm, scl_hbm, ocl_hbm, prd_hbm, sx_hbm, sy_hbm, ox_hbm, oy_hbm)
    cps = [pltpu.async_copy(src.at[pl.ds(base, RPW)],
                            int_buf.at[pl.ds(f * RPW, RPW)], sem_in)
           for f, src in enumerate(ins)]
    for cp in cps:
        cp.wait()

    def meta_group(g, _):
        off = g * 16
        bi = int_buf[pl.ds(0 * RPW + off, 16)]
        scl = int_buf[pl.ds(1 * RPW + off, 16)]
        ocl = int_buf[pl.ds(2 * RPW + off, 16)]
        sxi = int_buf[pl.ds(4 * RPW + off, 16)]
        syi = int_buf[pl.ds(5 * RPW + off, 16)]
        oxi = int_buf[pl.ds(6 * RPW + off, 16)]
        oyi = int_buf[pl.ds(7 * RPW + off, 16)]

        hidx[pl.ds(off, 16)] = ((bi * C + scl) * H + syi) * W + sxi
        hidx[pl.ds(RPW + off, 16)] = ((bi * C + ocl) * H + oyi) * W + oxi

        sxf = sxi.astype(jnp.float32)
        syf = syi.astype(jnp.float32)
        oxf = oxi.astype(jnp.float32)
        oyf = oyi.astype(jnp.float32)
        dx = oxf - sxf
        dy = oyf - syf
        n2 = dx * dx + dy * dy
        r = _rsqrt_f32(jnp.maximum(n2, np.float32(1.0)))
        uxb[pl.ds(off, 16)] = dx * r
        uyb[pl.ds(off, 16)] = dy * r
        validb[pl.ds(off, 16)] = jnp.where(n2 > np.float32(0.0),
                                           np.float32(1.0), np.float32(0.0))
        return 0

    lax.fori_loop(0, GROUPS, meta_group, 0)
    cp_h = pltpu.async_copy(hm_hbm.at[hidx], hval, sem_h)

    def build_group(g):
        off = g * 16
        bi = int_buf[pl.ds(0 * RPW + off, 16)]
        prd = int_buf[pl.ds(3 * RPW + off, 16)]
        sxf = int_buf[pl.ds(4 * RPW + off, 16)].astype(jnp.float32)
        syf = int_buf[pl.ds(5 * RPW + off, 16)].astype(jnp.float32)
        oxf = int_buf[pl.ds(6 * RPW + off, 16)].astype(jnp.float32)
        oyf = int_buf[pl.ds(7 * RPW + off, 16)].astype(jnp.float32)
        rbase = (bi * (2 * P) + 2 * prd) * HW
        ddx = sxf - oxf
        ddy = syf - oyf

        def build_row(r32, _):
            rowoff = g * GSZ + r32 * 128
            for k in range(4):
                s = r32 * 4 + k
                t = s.astype(jnp.float32) * INV_T
                px = _rint_idx(oxf + t * ddx)
                py = _rint_idx(oyf + t * ddy)
                i0 = rbase + py * W + px
                idx_buf[pl.ds(rowoff + k * 32, 16)] = i0
                idx_buf[pl.ds(rowoff + k * 32 + 16, 16)] = i0 + HW
            return 0

        lax.fori_loop(0, 32, build_row, 0)

    def reduce_group(g):
        off = g * 16
        ux = uxb[pl.ds(off, 16)]
        uy = uyb[pl.ds(off, 16)]

        def red_row(r32, acc):
            rowoff = g * GSZ + r32 * 128
            for k in range(4):
                g0 = g_buf[pl.ds(rowoff + k * 32, 16)]
                g1 = g_buf[pl.ds(rowoff + k * 32 + 16, 16)]
                acc = acc + g0 * ux + g1 * uy
            return acc

        acc = lax.fori_loop(0, 32, red_row, jnp.zeros((16,), jnp.float32))
        integb[pl.ds(off, 16)] = jnp.minimum(
            jnp.maximum(acc * INV_S, np.float32(0.0)), np.float32(1.0))

    raf_cps = []
    for g in range(GROUPS):
        build_group(g)
        raf_cps.append(pltpu.async_copy(
            raf_hbm.at[idx_buf.at[pl.ds(g * GSZ, GSZ)]],
            g_buf.at[pl.ds(g * GSZ, GSZ)], sem_r))

    cp_h.wait()

    def so_group(g, _):
        off = g * 16
        sob[pl.ds(off, 16)] = hval[pl.ds(off, 16)] * hval[pl.ds(RPW + off, 16)]
        return 0

    lax.fori_loop(0, GROUPS, so_group, 0)

    for g in range(GROUPS):
        raf_cps[g].wait()
        reduce_group(g)

    pltpu.sync_copy(integb, integ_hbm.at[pl.ds(base, RPW)])
    pltpu.sync_copy(sob, so_hbm.at[pl.ds(base, RPW)])
    pltpu.sync_copy(validb, valid_hbm.at[pl.ds(base, RPW)])


_sc_compute = functools.partial(
    pl.kernel,
    out_type=(jax.ShapeDtypeStruct((R,), jnp.float32),
              jax.ShapeDtypeStruct((R,), jnp.float32),
              jax.ShapeDtypeStruct((R,), jnp.float32)),
    mesh=plsc.VectorSubcoreMesh(core_axis_name="c", subcore_axis_name="s",
                                num_cores=NC, num_subcores=NS),
    scratch_types=[
        pltpu.VMEM((8 * RPW,), jnp.int32),
        pltpu.VMEM((16384,), jnp.int32),
        pltpu.VMEM((16384,), jnp.float32),
        pltpu.VMEM((2 * RPW,), jnp.int32),
        pltpu.VMEM((2 * RPW,), jnp.float32),
        pltpu.VMEM((RPW,), jnp.float32),
        pltpu.VMEM((RPW,), jnp.float32),
        pltpu.VMEM((RPW,), jnp.float32),
        pltpu.VMEM((RPW,), jnp.float32),
        pltpu.VMEM((RPW,), jnp.float32),
        pltpu.SemaphoreType.DMA,
        pltpu.SemaphoreType.DMA,
        pltpu.SemaphoreType.DMA,
    ],
)(_sc_body)


def _loss_body(so_col, integ_row, valid_col, valid_row, out_ref):
    mj = valid_row[...]
    b_row = jnp.where(mj > np.float32(0.0),
                      jnp.log(integ_row[...]), NEG_INF)

    def body(i, acc):
        so8 = so_col[pl.ds(i * 8, 8), :]
        mi = valid_col[pl.ds(i * 8, 8), :]
        a8 = jnp.where(mi > np.float32(0.0),
                       jnp.log(so8) - LOG_EPS, NEG_INF)
        return acc + jnp.maximum(a8 + b_row, np.float32(0.0))

    acc = lax.fori_loop(0, R // 8, body,
                        jnp.zeros((8, R), jnp.float32))
    nv = jnp.sum(mj)
    s = jnp.sum(acc) + LOG_EPS * nv * nv
    loss = -s / jnp.maximum(nv * nv, np.float32(1.0)) * LOSS_W
    out_ref[...] = loss.reshape(1, 1)


_loss_call = pl.pallas_call(
    _loss_body,
    out_shape=jax.ShapeDtypeStruct((1, 1), jnp.float32),
)


def kernel(rafs, heatmaps, batch_inds, subj_classes, obj_classes,
           subj_centers, obj_centers, predicates):
    raf_flat = jnp.clip(rafs, -1.0, 1.0).reshape(-1)
    hm_flat = heatmaps.reshape(-1)
    integ, so, valid = _sc_compute(
        raf_flat, hm_flat, batch_inds, subj_classes, obj_classes, predicates,
        subj_centers[:, 0], subj_centers[:, 1],
        obj_centers[:, 0], obj_centers[:, 1])
    loss = _loss_call(so.reshape(R, 1), integ.reshape(1, R),
                      valid.reshape(R, 1), valid.reshape(1, R))
    return loss.reshape(())


# speedup vs baseline: 1.2433x; 1.2433x over previous
import functools

import jax
import jax.numpy as jnp
import numpy as np
from jax import lax
from jax.experimental import pallas as pl
from jax.experimental.pallas import tpu as pltpu
from jax.experimental.pallas import tpu_sc as plsc

B = 4
P = 50
H = 200
W = 200
C = 80
R = 2048
S = 128

NC = 2
NS = 16
NW = NC * NS
RPW = R // NW
GROUPS = RPW // 16
GSZ = 16 * S * 2

WP = 256
HWP = H * WP
MAGIC = np.float32(2.0 ** 23)
INV_T = np.float32(1.0 / (S - 1))
INV_S = np.float32(1.0 / S)
RSQRT_MAGIC = np.int32(0x5F3759DF)
LOG_EPS = np.float32(np.log(np.float32(1e-12)))
NEG_INF = np.float32(-np.inf)
LOSS_W = np.float32(0.1)


def _rsqrt_f32(x):
    i = lax.bitcast_convert_type(x, jnp.int32)
    i = RSQRT_MAGIC - lax.shift_right_logical(i, 1)
    y = lax.bitcast_convert_type(i, jnp.float32)
    for _ in range(3):
        y = y * (np.float32(1.5) - np.float32(0.5) * x * y * y)
    return y


def _rint_idx(x):
    r = (x + MAGIC) - MAGIC
    r = jnp.minimum(jnp.maximum(r, np.float32(0.0)), np.float32(199.0))
    return r.astype(jnp.int32)


def _sc_body(raf_hbm, hm_hbm, bi_hbm, scl_hbm, ocl_hbm, prd_hbm,
             sx_hbm, sy_hbm, ox_hbm, oy_hbm,
             integ_hbm, so_hbm, valid_hbm,
             int_buf, idx_buf, g_buf, hidx, hval,
             uxb, uyb, integb, sob, validb, sem_in, sem_r, sem_h):
    wid = lax.axis_index("c") * NS + lax.axis_index("s")
    base = wid * RPW

    ins = (bi_hbm, scl_hbm, ocl_hbm, prd_hbm, sx_hbm, sy_hbm, ox_hbm, oy_hbm)
    cps = [pltpu.async_copy(src.at[pl.ds(base, RPW)],
                            int_buf.at[pl.ds(f * RPW, RPW)], sem_in)
           for f, src in enumerate(ins)]
    for cp in cps:
        cp.wait()

    def meta_group(g, _):
        off = g * 16
        bi = int_buf[pl.ds(0 * RPW + off, 16)]
        scl = int_buf[pl.ds(1 * RPW + off, 16)]
        ocl = int_buf[pl.ds(2 * RPW + off, 16)]
        sxi = int_buf[pl.ds(4 * RPW + off, 16)]
        syi = int_buf[pl.ds(5 * RPW + off, 16)]
        oxi = int_buf[pl.ds(6 * RPW + off, 16)]
        oyi = int_buf[pl.ds(7 * RPW + off, 16)]

        hidx[pl.ds(off, 16)] = ((bi * C + scl) * H + syi) * WP + sxi
        hidx[pl.ds(RPW + off, 16)] = ((bi * C + ocl) * H + oyi) * WP + oxi

        sxf = sxi.astype(jnp.float32)
        syf = syi.astype(jnp.float32)
        oxf = oxi.astype(jnp.float32)
        oyf = oyi.astype(jnp.float32)
        dx = oxf - sxf
        dy = oyf - syf
        n2 = dx * dx + dy * dy
        r = _rsqrt_f32(jnp.maximum(n2, np.float32(1.0)))
        uxb[pl.ds(off, 16)] = dx * r
        uyb[pl.ds(off, 16)] = dy * r
        validb[pl.ds(off, 16)] = jnp.where(n2 > np.float32(0.0),
                                           np.float32(1.0), np.float32(0.0))
        return 0

    lax.fori_loop(0, GROUPS, meta_group, 0)
    cp_h = pltpu.async_copy(hm_hbm.at[hidx], hval, sem_h)

    def build_group(g):
        off = g * 16
        bi = int_buf[pl.ds(0 * RPW + off, 16)]
        prd = int_buf[pl.ds(3 * RPW + off, 16)]
        sxf = int_buf[pl.ds(4 * RPW + off, 16)].astype(jnp.float32)
        syf = int_buf[pl.ds(5 * RPW + off, 16)].astype(jnp.float32)
        oxf = int_buf[pl.ds(6 * RPW + off, 16)].astype(jnp.float32)
        oyf = int_buf[pl.ds(7 * RPW + off, 16)].astype(jnp.float32)
        rbase = (bi * (2 * P) + 2 * prd) * HWP
        ddx = sxf - oxf
        ddy = syf - oyf

        def build_row(r32, _):
            rowoff = g * GSZ + r32 * 128
            for k in range(4):
                s = r32 * 4 + k
                t = s.astype(jnp.float32) * INV_T
                px = _rint_idx(oxf + t * ddx)
                py = _rint_idx(oyf + t * ddy)
                i0 = rbase + py * WP + px
                idx_buf[pl.ds(rowoff + k * 32, 16)] = i0
                idx_buf[pl.ds(rowoff + k * 32 + 16, 16)] = i0 + HWP
            return 0

        lax.fori_loop(0, 32, build_row, 0)

    def reduce_group(g):
        off = g * 16
        ux = uxb[pl.ds(off, 16)]
        uy = uyb[pl.ds(off, 16)]

        def red_row(r32, acc):
            rowoff = g * GSZ + r32 * 128
            for k in range(4):
                g0 = g_buf[pl.ds(rowoff + k * 32, 16)]
                g1 = g_buf[pl.ds(rowoff + k * 32 + 16, 16)]
                g0 = jnp.minimum(jnp.maximum(g0, np.float32(-1.0)), np.float32(1.0))
                g1 = jnp.minimum(jnp.maximum(g1, np.float32(-1.0)), np.float32(1.0))
                acc = acc + g0 * ux + g1 * uy
            return acc

        acc = lax.fori_loop(0, 32, red_row, jnp.zeros((16,), jnp.float32))
        integb[pl.ds(off, 16)] = jnp.minimum(
            jnp.maximum(acc * INV_S, np.float32(0.0)), np.float32(1.0))

    raf_cps = []
    for g in range(GROUPS):
        build_group(g)
        raf_cps.append(pltpu.async_copy(
            raf_hbm.at[idx_buf.at[pl.ds(g * GSZ, GSZ)]],
            g_buf.at[pl.ds(g * GSZ, GSZ)], sem_r))

    cp_h.wait()

    def so_group(g, _):
        off = g * 16
        sob[pl.ds(off, 16)] = hval[pl.ds(off, 16)] * hval[pl.ds(RPW + off, 16)]
        return 0

    lax.fori_loop(0, GROUPS, so_group, 0)

    for g in range(GROUPS):
        raf_cps[g].wait()
        reduce_group(g)

    pltpu.sync_copy(integb, integ_hbm.at[pl.ds(base, RPW)])
    pltpu.sync_copy(sob, so_hbm.at[pl.ds(base, RPW)])
    pltpu.sync_copy(validb, valid_hbm.at[pl.ds(base, RPW)])


_sc_compute = functools.partial(
    pl.kernel,
    out_type=(jax.ShapeDtypeStruct((R,), jnp.float32),
              jax.ShapeDtypeStruct((R,), jnp.float32),
              jax.ShapeDtypeStruct((R,), jnp.float32)),
    mesh=plsc.VectorSubcoreMesh(core_axis_name="c", subcore_axis_name="s",
                                num_cores=NC, num_subcores=NS),
    scratch_types=[
        pltpu.VMEM((8 * RPW,), jnp.int32),
        pltpu.VMEM((16384,), jnp.int32),
        pltpu.VMEM((16384,), jnp.float32),
        pltpu.VMEM((2 * RPW,), jnp.int32),
        pltpu.VMEM((2 * RPW,), jnp.float32),
        pltpu.VMEM((RPW,), jnp.float32),
        pltpu.VMEM((RPW,), jnp.float32),
        pltpu.VMEM((RPW,), jnp.float32),
        pltpu.VMEM((RPW,), jnp.float32),
        pltpu.VMEM((RPW,), jnp.float32),
        pltpu.SemaphoreType.DMA,
        pltpu.SemaphoreType.DMA,
        pltpu.SemaphoreType.DMA,
    ],
)(_sc_body)


def _loss_body(so_col, integ_row, valid_col, valid_row, out_ref):
    mj = valid_row[...]
    b_row = jnp.where(mj > np.float32(0.0),
                      jnp.log(integ_row[...]), NEG_INF)

    def body(i, acc):
        so16 = so_col[pl.ds(i * 16, 16), :]
        mi = valid_col[pl.ds(i * 16, 16), :]
        a16 = jnp.where(mi > np.float32(0.0),
                        jnp.log(so16) - LOG_EPS, NEG_INF)
        return acc + jnp.maximum(a16 + b_row, np.float32(0.0))

    acc = lax.fori_loop(0, R // 16, body,
                        jnp.zeros((16, R), jnp.float32))
    nv = jnp.sum(mj)
    s = jnp.sum(acc) + LOG_EPS * nv * nv
    loss = -s / jnp.maximum(nv * nv, np.float32(1.0)) * LOSS_W
    out_ref[...] = loss.reshape(1, 1)


_loss_call = pl.pallas_call(
    _loss_body,
    out_shape=jax.ShapeDtypeStruct((1, 1), jnp.float32),
)


def kernel(rafs, heatmaps, batch_inds, subj_classes, obj_classes,
           subj_centers, obj_centers, predicates):
    pad = ((0, 0), (0, 0), (0, 0), (0, WP - W))
    raf_flat = jnp.pad(rafs, pad).reshape(-1)
    hm_flat = jnp.pad(heatmaps, pad).reshape(-1)
    integ, so, valid = _sc_compute(
        raf_flat, hm_flat, batch_inds, subj_classes, obj_classes, predicates,
        subj_centers[:, 0], subj_centers[:, 1],
        obj_centers[:, 0], obj_centers[:, 1])
    loss = _loss_call(so.reshape(R, 1), integ.reshape(1, R),
                      valid.reshape(R, 1), valid.reshape(1, R))
    return loss.reshape(())
